# baseline (speedup 1.0000x reference)
import jax
import jax.numpy as jnp
from jax.experimental import pallas as pl
from jax.experimental.pallas import tpu as pltpu
from jax.experimental.pallas import tpu_sc as plsc

T = 512
D = 2048
F = 7168
E = 8
EPAD = 128
BLK_F = 256
NF = F // BLK_F
BLK_M = 128
NB_MAX = 16
R_PAD = NB_MAX * BLK_M


def _gate_body(logits_ref, pos0_ref, pos1_ref, rw1_ref, rw2_ref,
               nblk_ref, bstart_ref):
    i32 = jnp.int32
    lane = jax.lax.broadcasted_iota(i32, (T, EPAD), 1)
    valid = lane < E
    neg_inf = jnp.float32(-jnp.inf)
    logits = jnp.where(valid, logits_ref[...], neg_inf)
    lmax = jnp.max(logits, axis=1, keepdims=True)
    unnorm = jnp.exp(logits - lmax)
    p = unnorm / jnp.sum(unnorm, axis=1, keepdims=True)
    p = jnp.where(valid, p, neg_inf)
    m1 = jnp.max(p, axis=1, keepdims=True)
    i1 = jnp.min(jnp.where(p == m1, lane, EPAD), axis=1, keepdims=True)
    oh1 = lane == i1
    p2 = jnp.where(oh1, neg_inf, p)
    m2 = jnp.max(p2, axis=1, keepdims=True)
    i2 = jnp.min(jnp.where(p2 == m2, lane, EPAD), axis=1, keepdims=True)
    oh2 = lane == i2
    denom = m1 + m2
    rw1_ref[...] = m1 / denom
    rw2_ref[...] = m2 / denom

    A = oh1.astype(jnp.bfloat16) + oh2.astype(jnp.bfloat16)
    ti = jax.lax.broadcasted_iota(i32, (T, T), 0)
    tj = jax.lax.broadcasted_iota(i32, (T, T), 1)
    ltri = (tj < ti).astype(jnp.bfloat16)
    cex = jax.lax.dot_general(
        ltri, A, (((1,), (0,)), ((), ())),
        preferred_element_type=jnp.float32)
    rank1 = jnp.sum(jnp.where(oh1, cex, 0.0), axis=1, keepdims=True)
    rank2 = jnp.sum(jnp.where(oh2, cex, 0.0), axis=1, keepdims=True)
    counts = jnp.sum(A.astype(jnp.float32), axis=0, keepdims=True)
    nblk = jnp.floor((counts + (BLK_M - 1)) / BLK_M)
    nblk_ref[...] = nblk.astype(i32)
    ei = jax.lax.broadcasted_iota(i32, (EPAD, EPAD), 0)
    ej = jax.lax.broadcasted_iota(i32, (EPAD, EPAD), 1)
    utri = (ei < ej).astype(jnp.bfloat16)
    bstart = jax.lax.dot_general(
        nblk.astype(jnp.bfloat16), utri, (((1,), (0,)), ((), ())),
        preferred_element_type=jnp.float32)
    bstart_ref[...] = bstart.astype(i32)
    base1 = jnp.sum(jnp.where(oh1, bstart, 0.0), axis=1, keepdims=True)
    base2 = jnp.sum(jnp.where(oh2, bstart, 0.0), axis=1, keepdims=True)
    pos0_ref[...] = (base1 * BLK_M + rank1).astype(i32)
    pos1_ref[...] = (base2 * BLK_M + rank2).astype(i32)


def _gate(logits_pad):
    return pl.pallas_call(
        _gate_body,
        out_shape=[
            jax.ShapeDtypeStruct((T, 1), jnp.int32),
            jax.ShapeDtypeStruct((T, 1), jnp.int32),
            jax.ShapeDtypeStruct((T, 1), jnp.float32),
            jax.ShapeDtypeStruct((T, 1), jnp.float32),
            jax.ShapeDtypeStruct((1, EPAD), jnp.int32),
            jax.ShapeDtypeStruct((1, EPAD), jnp.int32),
        ],
    )(logits_pad)


def _ffn_body(nblk_ref, bstart_ref, x_ref, pos0_ref, pos1_ref,
              rw1_ref, rw2_ref, w1_ref, w3_ref, w2_ref, o_ref):
    e = pl.program_id(0)
    f = pl.program_id(1)
    w1 = w1_ref[0].astype(jnp.bfloat16)
    w3 = w3_ref[0].astype(jnp.bfloat16)
    w2 = w2_ref[0].astype(jnp.bfloat16)
    x = x_ref[...]
    pos0 = pos0_ref[...]
    pos1 = pos1_ref[...]

    def blk(jdx, carry):
        base = (bstart_ref[e] + jdx) * BLK_M
        slot = jax.lax.broadcasted_iota(jnp.int32, (T, BLK_M), 1) + base
        s1 = pos0 == slot
        s2 = pos1 == slot
        sel = (s1 | s2).astype(jnp.bfloat16)
        xs = jax.lax.dot_general(
            sel, x, (((0,), (0,)), ((), ())),
            preferred_element_type=jnp.float32).astype(jnp.bfloat16)
        roww = jax.lax.dot_general(
            s1.astype(jnp.float32), rw1_ref[...], (((0,), (0,)), ((), ())),
            preferred_element_type=jnp.float32) + jax.lax.dot_general(
            s2.astype(jnp.float32), rw2_ref[...], (((0,), (0,)), ((), ())),
            preferred_element_type=jnp.float32)
        h1 = jax.lax.dot_general(
            xs, w1, (((1,), (0,)), ((), ())),
            preferred_element_type=jnp.float32)
        h3 = jax.lax.dot_general(
            xs, w3, (((1,), (0,)), ((), ())),
            preferred_element_type=jnp.float32)
        g = (h1 * jax.lax.logistic(h1)) * h3
        g = g * roww
        contrib = jax.lax.dot_general(
            g.astype(jnp.bfloat16), w2, (((1,), (0,)), ((), ())),
            preferred_element_type=jnp.float32)

        @pl.when(f == 0)
        def _set():
            o_ref[pl.ds(base, BLK_M), :] = contrib

        @pl.when(f != 0)
        def _add():
            o_ref[pl.ds(base, BLK_M), :] += contrib

        return carry

    jax.lax.fori_loop(0, nblk_ref[e], blk, 0)


def _ffn(x_bf16, pos0, pos1, rw1, rw2, w1, w3, w2, nblk, bstart):
    grid_spec = pltpu.PrefetchScalarGridSpec(
        num_scalar_prefetch=2,
        grid=(E, NF),
        in_specs=[
            pl.BlockSpec((T, D), lambda e, f, nb, bs: (0, 0)),
            pl.BlockSpec((T, 1), lambda e, f, nb, bs: (0, 0)),
            pl.BlockSpec((T, 1), lambda e, f, nb, bs: (0, 0)),
            pl.BlockSpec((T, 1), lambda e, f, nb, bs: (0, 0)),
            pl.BlockSpec((T, 1), lambda e, f, nb, bs: (0, 0)),
            pl.BlockSpec((1, D, BLK_F), lambda e, f, nb, bs: (e, 0, f)),
            pl.BlockSpec((1, D, BLK_F), lambda e, f, nb, bs: (e, 0, f)),
            pl.BlockSpec((1, BLK_F, D), lambda e, f, nb, bs: (e, f, 0)),
        ],
        out_specs=pl.BlockSpec((R_PAD, D), lambda e, f, nb, bs: (0, 0)),
    )
    return pl.pallas_call(
        _ffn_body,
        grid_spec=grid_spec,
        out_shape=jax.ShapeDtypeStruct((R_PAD, D), jnp.float32),
        compiler_params=pltpu.CompilerParams(
            dimension_semantics=("arbitrary", "arbitrary")),
    )(nblk, bstart, x_bf16, pos0, pos1, rw1, rw2, w1, w3, w2)


_vector_mesh = None


def _get_vector_mesh():
    global _vector_mesh
    if _vector_mesh is None:
        _vector_mesh = plsc.VectorSubcoreMesh(
            core_axis_name="core", subcore_axis_name="subcore")
    return _vector_mesh


def _sc_combine(rows, pos0, pos1):
    @pl.kernel(out_type=jax.ShapeDtypeStruct((T, D), jnp.float32),
               mesh=_get_vector_mesh(),
               scratch_types=[pltpu.VMEM((T,), jnp.int32),
                              pltpu.VMEM((T,), jnp.int32),
                              pltpu.VMEM((16, D), jnp.float32),
                              pltpu.VMEM((16, D), jnp.float32)])
    def k(r_hbm, i0_hbm, i1_hbm, o_hbm, i0_v, i1_v, buf_a, buf_b):
        wid = (jax.lax.axis_index("subcore") * 2
               + jax.lax.axis_index("core"))
        pltpu.sync_copy(i0_hbm, i0_v)
        pltpu.sync_copy(i1_hbm, i1_v)
        pltpu.sync_copy(r_hbm.at[i0_v.at[pl.ds(wid * 16, 16)]], buf_a)
        pltpu.sync_copy(r_hbm.at[i1_v.at[pl.ds(wid * 16, 16)]], buf_b)

        @pl.loop(0, 16)
        def _(r):
            @pl.loop(0, D, step=16)
            def _(c):
                buf_a.at[r, pl.ds(c, 16)][...] = (
                    buf_a.at[r, pl.ds(c, 16)][...]
                    + buf_b.at[r, pl.ds(c, 16)][...])

        pltpu.sync_copy(buf_a, o_hbm.at[pl.ds(wid * 16, 16), :])

    return k(rows, pos0, pos1)


@jax.jit
def kernel(hidden_states, Wg, W1, W2, W3):
    router_logits = hidden_states @ Wg
    logits_pad = jnp.pad(router_logits, ((0, 0), (0, EPAD - E)),
                         constant_values=-jnp.inf)
    pos0, pos1, rw1, rw2, nblk, bstart = _gate(logits_pad)
    out_rows = _ffn(hidden_states.astype(jnp.bfloat16), pos0, pos1, rw1, rw2,
                    W1, W3, W2, nblk[0, :E], bstart[0, :E])
    return _sc_combine(out_rows, pos0.reshape(T), pos1.reshape(T))


# speedup vs baseline: 1.3199x; 1.3199x over previous
import jax
import jax.numpy as jnp
from jax.experimental import pallas as pl
from jax.experimental.pallas import tpu as pltpu
from jax.experimental.pallas import tpu_sc as plsc

T = 512
D = 2048
F = 7168
E = 8
EPAD = 128
BLK_F = 512
NF = F // BLK_F
BLK_M = 128
NB_MAX = 15
R_PAD = NB_MAX * BLK_M


def _gate_body(logits_ref, pos0_ref, pos1_ref, rw1_ref, rw2_ref,
               nblk_ref, bstart_ref):
    i32 = jnp.int32
    lane = jax.lax.broadcasted_iota(i32, (T, EPAD), 1)
    valid = lane < E
    neg_inf = jnp.float32(-jnp.inf)
    logits = jnp.where(valid, logits_ref[...], neg_inf)
    lmax = jnp.max(logits, axis=1, keepdims=True)
    unnorm = jnp.exp(logits - lmax)
    p = unnorm / jnp.sum(unnorm, axis=1, keepdims=True)
    p = jnp.where(valid, p, neg_inf)
    m1 = jnp.max(p, axis=1, keepdims=True)
    i1 = jnp.min(jnp.where(p == m1, lane, EPAD), axis=1, keepdims=True)
    oh1 = lane == i1
    p2 = jnp.where(oh1, neg_inf, p)
    m2 = jnp.max(p2, axis=1, keepdims=True)
    i2 = jnp.min(jnp.where(p2 == m2, lane, EPAD), axis=1, keepdims=True)
    oh2 = lane == i2
    denom = m1 + m2
    rw1_ref[...] = m1 / denom
    rw2_ref[...] = m2 / denom

    A = oh1.astype(jnp.bfloat16) + oh2.astype(jnp.bfloat16)
    ti = jax.lax.broadcasted_iota(i32, (T, T), 0)
    tj = jax.lax.broadcasted_iota(i32, (T, T), 1)
    ltri = (tj < ti).astype(jnp.bfloat16)
    cex = jax.lax.dot_general(
        ltri, A, (((1,), (0,)), ((), ())),
        preferred_element_type=jnp.float32)
    rank1 = jnp.sum(jnp.where(oh1, cex, 0.0), axis=1, keepdims=True)
    rank2 = jnp.sum(jnp.where(oh2, cex, 0.0), axis=1, keepdims=True)
    counts = jnp.sum(A.astype(jnp.float32), axis=0, keepdims=True)
    nblk = jnp.floor((counts + (BLK_M - 1)) / BLK_M)
    nblk_ref[...] = nblk.astype(i32)
    ei = jax.lax.broadcasted_iota(i32, (EPAD, EPAD), 0)
    ej = jax.lax.broadcasted_iota(i32, (EPAD, EPAD), 1)
    utri = (ei < ej).astype(jnp.bfloat16)
    bstart = jax.lax.dot_general(
        nblk.astype(jnp.bfloat16), utri, (((1,), (0,)), ((), ())),
        preferred_element_type=jnp.float32)
    bstart_ref[...] = bstart.astype(i32)
    base1 = jnp.sum(jnp.where(oh1, bstart, 0.0), axis=1, keepdims=True)
    base2 = jnp.sum(jnp.where(oh2, bstart, 0.0), axis=1, keepdims=True)
    pos0_ref[...] = (base1 * BLK_M + rank1).astype(i32)
    pos1_ref[...] = (base2 * BLK_M + rank2).astype(i32)


def _gate(logits_pad):
    return pl.pallas_call(
        _gate_body,
        out_shape=[
            jax.ShapeDtypeStruct((T, 1), jnp.int32),
            jax.ShapeDtypeStruct((T, 1), jnp.int32),
            jax.ShapeDtypeStruct((T, 1), jnp.float32),
            jax.ShapeDtypeStruct((T, 1), jnp.float32),
            jax.ShapeDtypeStruct((1, EPAD), jnp.int32),
            jax.ShapeDtypeStruct((1, EPAD), jnp.int32),
        ],
    )(logits_pad)


def _ffn_body(nblk_ref, bstart_ref, x_ref, pos0_ref, pos1_ref,
              rw1_ref, rw2_ref, w1_ref, w3_ref, w2_ref, o_ref):
    e = pl.program_id(0)
    f = pl.program_id(1)
    w1 = w1_ref[0].astype(jnp.bfloat16)
    w3 = w3_ref[0].astype(jnp.bfloat16)
    w2 = w2_ref[0].astype(jnp.bfloat16)
    x = x_ref[...]
    pos0 = pos0_ref[...]
    pos1 = pos1_ref[...]

    def blk(jdx, carry):
        base = (bstart_ref[e] + jdx) * BLK_M
        slot = jax.lax.broadcasted_iota(jnp.int32, (T, BLK_M), 1) + base
        s1 = pos0 == slot
        s2 = pos1 == slot
        sel = (s1 | s2).astype(jnp.bfloat16)
        xs = jax.lax.dot_general(
            sel, x, (((0,), (0,)), ((), ())),
            preferred_element_type=jnp.float32).astype(jnp.bfloat16)
        roww = jax.lax.dot_general(
            s1.astype(jnp.float32), rw1_ref[...], (((0,), (0,)), ((), ())),
            preferred_element_type=jnp.float32) + jax.lax.dot_general(
            s2.astype(jnp.float32), rw2_ref[...], (((0,), (0,)), ((), ())),
            preferred_element_type=jnp.float32)
        h1 = jax.lax.dot_general(
            xs, w1, (((1,), (0,)), ((), ())),
            preferred_element_type=jnp.float32)
        h3 = jax.lax.dot_general(
            xs, w3, (((1,), (0,)), ((), ())),
            preferred_element_type=jnp.float32)
        g = (h1 * jax.lax.logistic(h1)) * h3
        g = g * roww
        contrib = jax.lax.dot_general(
            g.astype(jnp.bfloat16), w2, (((1,), (0,)), ((), ())),
            preferred_element_type=jnp.float32)

        @pl.when(f == 0)
        def _set():
            o_ref[pl.ds(base, BLK_M), :] = contrib

        @pl.when(f != 0)
        def _add():
            o_ref[pl.ds(base, BLK_M), :] += contrib

        return carry

    jax.lax.fori_loop(0, nblk_ref[e], blk, 0)


def _ffn(x_bf16, pos0, pos1, rw1, rw2, w1, w3, w2, nblk, bstart):
    grid_spec = pltpu.PrefetchScalarGridSpec(
        num_scalar_prefetch=2,
        grid=(E, NF),
        in_specs=[
            pl.BlockSpec((T, D), lambda e, f, nb, bs: (0, 0)),
            pl.BlockSpec((T, 1), lambda e, f, nb, bs: (0, 0)),
            pl.BlockSpec((T, 1), lambda e, f, nb, bs: (0, 0)),
            pl.BlockSpec((T, 1), lambda e, f, nb, bs: (0, 0)),
            pl.BlockSpec((T, 1), lambda e, f, nb, bs: (0, 0)),
            pl.BlockSpec((1, D, BLK_F), lambda e, f, nb, bs: (e, 0, f)),
            pl.BlockSpec((1, D, BLK_F), lambda e, f, nb, bs: (e, 0, f)),
            pl.BlockSpec((1, BLK_F, D), lambda e, f, nb, bs: (e, f, 0)),
        ],
        out_specs=pl.BlockSpec((R_PAD, D), lambda e, f, nb, bs: (0, 0)),
    )
    return pl.pallas_call(
        _ffn_body,
        grid_spec=grid_spec,
        out_shape=jax.ShapeDtypeStruct((R_PAD, D), jnp.float32),
        compiler_params=pltpu.CompilerParams(
            dimension_semantics=("arbitrary", "arbitrary")),
    )(nblk, bstart, x_bf16, pos0, pos1, rw1, rw2, w1, w3, w2)


_vector_mesh = None


def _get_vector_mesh():
    global _vector_mesh
    if _vector_mesh is None:
        _vector_mesh = plsc.VectorSubcoreMesh(
            core_axis_name="core", subcore_axis_name="subcore")
    return _vector_mesh


def _sc_combine(rows, pos0, pos1):
    @pl.kernel(out_type=jax.ShapeDtypeStruct((T, D), jnp.float32),
               mesh=_get_vector_mesh(),
               scratch_types=[pltpu.VMEM((T,), jnp.int32),
                              pltpu.VMEM((T,), jnp.int32),
                              pltpu.VMEM((16, D), jnp.float32),
                              pltpu.VMEM((16, D), jnp.float32)])
    def k(r_hbm, i0_hbm, i1_hbm, o_hbm, i0_v, i1_v, buf_a, buf_b):
        wid = (jax.lax.axis_index("subcore") * 2
               + jax.lax.axis_index("core"))
        pltpu.sync_copy(i0_hbm, i0_v)
        pltpu.sync_copy(i1_hbm, i1_v)
        pltpu.sync_copy(r_hbm.at[i0_v.at[pl.ds(wid * 16, 16)]], buf_a)
        pltpu.sync_copy(r_hbm.at[i1_v.at[pl.ds(wid * 16, 16)]], buf_b)

        @pl.loop(0, 16)
        def _(r):
            @pl.loop(0, D, step=16)
            def _(c):
                buf_a.at[r, pl.ds(c, 16)][...] = (
                    buf_a.at[r, pl.ds(c, 16)][...]
                    + buf_b.at[r, pl.ds(c, 16)][...])

        pltpu.sync_copy(buf_a, o_hbm.at[pl.ds(wid * 16, 16), :])

    return k(rows, pos0, pos1)


@jax.jit
def kernel(hidden_states, Wg, W1, W2, W3):
    router_logits = hidden_states @ Wg
    logits_pad = jnp.pad(router_logits, ((0, 0), (0, EPAD - E)),
                         constant_values=-jnp.inf)
    pos0, pos1, rw1, rw2, nblk, bstart = _gate(logits_pad)
    out_rows = _ffn(hidden_states.astype(jnp.bfloat16), pos0, pos1, rw1, rw2,
                    W1, W3, W2, nblk[0, :E], bstart[0, :E])
    return _sc_combine(out_rows, pos0.reshape(T), pos1.reshape(T))


# speedup vs baseline: 1.3201x; 1.0002x over previous
import jax
import jax.numpy as jnp
from jax.experimental import pallas as pl
from jax.experimental.pallas import tpu as pltpu
from jax.experimental.pallas import tpu_sc as plsc

T = 512
D = 2048
F = 7168
E = 8
EPAD = 128
BLK_F = 512
NF = F // BLK_F
BLK_M = 128
NB_MAX = 15
R_PAD = NB_MAX * BLK_M


def _gate_body(logits_ref, x_ref, pos0_ref, pos1_ref, rw1_ref, rw2_ref,
               nblk_ref, bstart_ref, xbf_ref):
    i32 = jnp.int32
    xbf_ref[...] = x_ref[...].astype(jnp.bfloat16)
    lane = jax.lax.broadcasted_iota(i32, (T, EPAD), 1)
    valid = lane < E
    neg_inf = jnp.float32(-jnp.inf)
    logits = jnp.where(valid, logits_ref[...], neg_inf)
    lmax = jnp.max(logits, axis=1, keepdims=True)
    unnorm = jnp.exp(logits - lmax)
    p = unnorm / jnp.sum(unnorm, axis=1, keepdims=True)
    p = jnp.where(valid, p, neg_inf)
    m1 = jnp.max(p, axis=1, keepdims=True)
    i1 = jnp.min(jnp.where(p == m1, lane, EPAD), axis=1, keepdims=True)
    oh1 = lane == i1
    p2 = jnp.where(oh1, neg_inf, p)
    m2 = jnp.max(p2, axis=1, keepdims=True)
    i2 = jnp.min(jnp.where(p2 == m2, lane, EPAD), axis=1, keepdims=True)
    oh2 = lane == i2
    denom = m1 + m2
    rw1_ref[...] = m1 / denom
    rw2_ref[...] = m2 / denom

    A = oh1.astype(jnp.bfloat16) + oh2.astype(jnp.bfloat16)
    ti = jax.lax.broadcasted_iota(i32, (T, T), 0)
    tj = jax.lax.broadcasted_iota(i32, (T, T), 1)
    ltri = (tj < ti).astype(jnp.bfloat16)
    cex = jax.lax.dot_general(
        ltri, A, (((1,), (0,)), ((), ())),
        preferred_element_type=jnp.float32)
    rank1 = jnp.sum(jnp.where(oh1, cex, 0.0), axis=1, keepdims=True)
    rank2 = jnp.sum(jnp.where(oh2, cex, 0.0), axis=1, keepdims=True)
    counts = jnp.sum(A.astype(jnp.float32), axis=0, keepdims=True)
    nblk = jnp.floor((counts + (BLK_M - 1)) / BLK_M)
    nblk_ref[...] = nblk.astype(i32)
    ei = jax.lax.broadcasted_iota(i32, (EPAD, EPAD), 0)
    ej = jax.lax.broadcasted_iota(i32, (EPAD, EPAD), 1)
    utri = (ei < ej).astype(jnp.bfloat16)
    bstart = jax.lax.dot_general(
        nblk.astype(jnp.bfloat16), utri, (((1,), (0,)), ((), ())),
        preferred_element_type=jnp.float32)
    bstart_ref[...] = bstart.astype(i32)
    base1 = jnp.sum(jnp.where(oh1, bstart, 0.0), axis=1, keepdims=True)
    base2 = jnp.sum(jnp.where(oh2, bstart, 0.0), axis=1, keepdims=True)
    pos0_ref[...] = (base1 * BLK_M + rank1).astype(i32)
    pos1_ref[...] = (base2 * BLK_M + rank2).astype(i32)


def _gate(logits_pad, x):
    return pl.pallas_call(
        _gate_body,
        out_shape=[
            jax.ShapeDtypeStruct((T, 1), jnp.int32),
            jax.ShapeDtypeStruct((T, 1), jnp.int32),
            jax.ShapeDtypeStruct((T, 1), jnp.float32),
            jax.ShapeDtypeStruct((T, 1), jnp.float32),
            jax.ShapeDtypeStruct((1, EPAD), jnp.int32),
            jax.ShapeDtypeStruct((1, EPAD), jnp.int32),
            jax.ShapeDtypeStruct((T, D), jnp.bfloat16),
        ],
    )(logits_pad, x)


def _ffn_body(nblk_ref, bstart_ref, x_ref, pos0_ref, pos1_ref,
              rw1_ref, rw2_ref, w1_ref, w3_ref, w2_ref, o_ref):
    e = pl.program_id(0)
    f = pl.program_id(1)
    w1 = w1_ref[0].astype(jnp.bfloat16)
    w3 = w3_ref[0].astype(jnp.bfloat16)
    w2 = w2_ref[0].astype(jnp.bfloat16)
    x = x_ref[...]
    pos0 = pos0_ref[...]
    pos1 = pos1_ref[...]

    def blk(jdx, carry):
        base = (bstart_ref[e] + jdx) * BLK_M
        slot = jax.lax.broadcasted_iota(jnp.int32, (T, BLK_M), 1) + base
        s1 = pos0 == slot
        s2 = pos1 == slot
        sel = (s1 | s2).astype(jnp.bfloat16)
        xs = jax.lax.dot_general(
            sel, x, (((0,), (0,)), ((), ())),
            preferred_element_type=jnp.float32).astype(jnp.bfloat16)
        roww = jax.lax.dot_general(
            s1.astype(jnp.float32), rw1_ref[...], (((0,), (0,)), ((), ())),
            preferred_element_type=jnp.float32) + jax.lax.dot_general(
            s2.astype(jnp.float32), rw2_ref[...], (((0,), (0,)), ((), ())),
            preferred_element_type=jnp.float32)
        h1 = jax.lax.dot_general(
            xs, w1, (((1,), (0,)), ((), ())),
            preferred_element_type=jnp.float32)
        h3 = jax.lax.dot_general(
            xs, w3, (((1,), (0,)), ((), ())),
            preferred_element_type=jnp.float32)
        g = (h1 * jax.lax.logistic(h1)) * h3
        g = g * roww
        contrib = jax.lax.dot_general(
            g.astype(jnp.bfloat16), w2, (((1,), (0,)), ((), ())),
            preferred_element_type=jnp.float32)

        @pl.when(f == 0)
        def _set():
            o_ref[pl.ds(base, BLK_M), :] = contrib

        @pl.when(f != 0)
        def _add():
            o_ref[pl.ds(base, BLK_M), :] += contrib

        return carry

    jax.lax.fori_loop(0, nblk_ref[e], blk, 0)


def _ffn(x_bf16, pos0, pos1, rw1, rw2, w1, w3, w2, nblk, bstart):
    grid_spec = pltpu.PrefetchScalarGridSpec(
        num_scalar_prefetch=2,
        grid=(E, NF),
        in_specs=[
            pl.BlockSpec((T, D), lambda e, f, nb, bs: (0, 0)),
            pl.BlockSpec((T, 1), lambda e, f, nb, bs: (0, 0)),
            pl.BlockSpec((T, 1), lambda e, f, nb, bs: (0, 0)),
            pl.BlockSpec((T, 1), lambda e, f, nb, bs: (0, 0)),
            pl.BlockSpec((T, 1), lambda e, f, nb, bs: (0, 0)),
            pl.BlockSpec((1, D, BLK_F), lambda e, f, nb, bs: (e, 0, f)),
            pl.BlockSpec((1, D, BLK_F), lambda e, f, nb, bs: (e, 0, f)),
            pl.BlockSpec((1, BLK_F, D), lambda e, f, nb, bs: (e, f, 0)),
        ],
        out_specs=pl.BlockSpec((R_PAD, D), lambda e, f, nb, bs: (0, 0)),
    )
    return pl.pallas_call(
        _ffn_body,
        grid_spec=grid_spec,
        out_shape=jax.ShapeDtypeStruct((R_PAD, D), jnp.float32),
        compiler_params=pltpu.CompilerParams(
            dimension_semantics=("arbitrary", "arbitrary")),
    )(nblk, bstart, x_bf16, pos0, pos1, rw1, rw2, w1, w3, w2)


_vector_mesh = None


def _get_vector_mesh():
    global _vector_mesh
    if _vector_mesh is None:
        _vector_mesh = plsc.VectorSubcoreMesh(
            core_axis_name="core", subcore_axis_name="subcore")
    return _vector_mesh


def _sc_combine(rows, pos0, pos1):
    @pl.kernel(out_type=jax.ShapeDtypeStruct((T, D), jnp.float32),
               mesh=_get_vector_mesh(),
               scratch_types=[pltpu.VMEM((T,), jnp.int32),
                              pltpu.VMEM((T,), jnp.int32),
                              pltpu.VMEM((16, D), jnp.float32),
                              pltpu.VMEM((16, D), jnp.float32)])
    def k(r_hbm, i0_hbm, i1_hbm, o_hbm, i0_v, i1_v, buf_a, buf_b):
        wid = (jax.lax.axis_index("subcore") * 2
               + jax.lax.axis_index("core"))
        pltpu.sync_copy(i0_hbm, i0_v)
        pltpu.sync_copy(i1_hbm, i1_v)
        pltpu.sync_copy(r_hbm.at[i0_v.at[pl.ds(wid * 16, 16)]], buf_a)
        pltpu.sync_copy(r_hbm.at[i1_v.at[pl.ds(wid * 16, 16)]], buf_b)

        @pl.loop(0, 16)
        def _(r):
            @pl.loop(0, D, step=16)
            def _(c):
                buf_a.at[r, pl.ds(c, 16)][...] = (
                    buf_a.at[r, pl.ds(c, 16)][...]
                    + buf_b.at[r, pl.ds(c, 16)][...])

        pltpu.sync_copy(buf_a, o_hbm.at[pl.ds(wid * 16, 16), :])

    return k(rows, pos0, pos1)


@jax.jit
def kernel(hidden_states, Wg, W1, W2, W3):
    router_logits = hidden_states @ Wg
    logits_pad = jnp.pad(router_logits, ((0, 0), (0, EPAD - E)),
                         constant_values=-jnp.inf)
    pos0, pos1, rw1, rw2, nblk, bstart, x_bf16 = _gate(
        logits_pad, hidden_states)
    out_rows = _ffn(x_bf16, pos0, pos1, rw1, rw2,
                    W1, W3, W2, nblk[0, :E], bstart[0, :E])
    return _sc_combine(out_rows, pos0.reshape(T), pos1.reshape(T))
